# baseline (speedup 1.0000x reference)
import functools
import jax
import jax.numpy as jnp
from jax import lax
from jax.experimental import pallas as pl
from jax.experimental.pallas import tpu as pltpu
from jax.experimental.pallas import tpu_sc as plsc

_ROWS = 16384
_COLS = 1024
_BLK = 512
_NC, _NS, _L = 2, 16, 16
_NW = _NC * _NS
_CHUNK = _ROWS // _NW
_TBL = 64

_sc_mesh = plsc.VectorSubcoreMesh(
    core_axis_name="c", subcore_axis_name="s", num_cores=_NC, num_subcores=_NS
)


@functools.partial(
    pl.kernel,
    out_type=[
        jax.ShapeDtypeStruct((_ROWS,), jnp.float32),
        jax.ShapeDtypeStruct((_ROWS,), jnp.float32),
    ],
    mesh=_sc_mesh,
    scratch_types=[
        pltpu.VMEM((_CHUNK,), jnp.int32),
        pltpu.VMEM((_TBL,), jnp.float32),
        pltpu.VMEM((_TBL,), jnp.float32),
        pltpu.VMEM((_CHUNK,), jnp.float32),
        pltpu.VMEM((_CHUNK,), jnp.float32),
    ],
    compiler_params=pltpu.CompilerParams(needs_layout_passes=False),
)
def _sc_gather(t_hbm, sa_hbm, sb_hbm, a_hbm, b_hbm, t_v, sa_v, sb_v, a_v, b_v):
    wid = lax.axis_index("s") * _NC + lax.axis_index("c")
    base = wid * _CHUNK
    pltpu.sync_copy(t_hbm.at[pl.ds(base, _CHUNK)], t_v)
    pltpu.sync_copy(sa_hbm, sa_v)
    pltpu.sync_copy(sb_hbm, sb_v)

    def body(i, carry):
        idx = t_v[pl.ds(i * _L, _L)]
        a_v[pl.ds(i * _L, _L)] = plsc.load_gather(sa_v, [idx])
        b_v[pl.ds(i * _L, _L)] = plsc.load_gather(sb_v, [idx])
        return carry

    lax.fori_loop(0, _CHUNK // _L, body, 0)
    pltpu.sync_copy(a_v, a_hbm.at[pl.ds(base, _CHUNK)])
    pltpu.sync_copy(b_v, b_hbm.at[pl.ds(base, _CHUNK)])


def _fma_kernel(a_ref, b_ref, x0_ref, eps_ref, o_ref):
    a = a_ref[0, 0, :]
    b = b_ref[0, 0, :]
    o_ref[...] = a[:, None] * x0_ref[...] + b[:, None] * eps_ref[...]


def kernel(x0, eps, t, alpha_hat):
    t32 = t.astype(jnp.int32)
    nb = alpha_hat.shape[0]
    sa = jnp.pad(jnp.sqrt(alpha_hat), (0, _TBL - nb))
    sb = jnp.pad(jnp.sqrt(1.0 - alpha_hat), (0, _TBL - nb))
    a, b = _sc_gather(t32, sa, sb)
    a3 = a.reshape(_ROWS // _BLK, 1, _BLK)
    b3 = b.reshape(_ROWS // _BLK, 1, _BLK)
    grid = (_ROWS // _BLK,)
    return pl.pallas_call(
        _fma_kernel,
        grid=grid,
        in_specs=[
            pl.BlockSpec((1, 1, _BLK), lambda i: (i, 0, 0)),
            pl.BlockSpec((1, 1, _BLK), lambda i: (i, 0, 0)),
            pl.BlockSpec((_BLK, _COLS), lambda i: (i, 0)),
            pl.BlockSpec((_BLK, _COLS), lambda i: (i, 0)),
        ],
        out_specs=pl.BlockSpec((_BLK, _COLS), lambda i: (i, 0)),
        out_shape=jax.ShapeDtypeStruct((_ROWS, _COLS), jnp.float32),
    )(a3, b3, x0, eps)


# speedup vs baseline: 1.3424x; 1.3424x over previous
import jax
import jax.numpy as jnp
from jax.experimental import pallas as pl

_ROWS = 16384
_COLS = 1024
_BLK = 1024
_TPAD = 128


def _fused_kernel(t_ref, sa_ref, sb_ref, x0_ref, eps_ref, o_ref):
    t_blk = t_ref[0, 0, :]
    cols = jax.lax.broadcasted_iota(jnp.int32, (t_blk.shape[0], _TPAD), 1)
    onehot = t_blk[:, None] == cols
    a = jnp.sum(jnp.where(onehot, sa_ref[0, :][None, :], 0.0), axis=1)
    b = jnp.sum(jnp.where(onehot, sb_ref[0, :][None, :], 0.0), axis=1)
    o_ref[...] = a[:, None] * x0_ref[...] + b[:, None] * eps_ref[...]


def kernel(x0, eps, t, alpha_hat):
    t32 = t.astype(jnp.int32).reshape(_ROWS // _BLK, 1, _BLK)
    nb = alpha_hat.shape[0]
    sa = jnp.sqrt(alpha_hat)
    sb = jnp.sqrt(1.0 - alpha_hat)
    pad = _TPAD - nb
    sa = jnp.pad(sa, (0, pad)).reshape(1, _TPAD)
    sb = jnp.pad(sb, (0, pad)).reshape(1, _TPAD)
    grid = (_ROWS // _BLK,)
    return pl.pallas_call(
        _fused_kernel,
        grid=grid,
        in_specs=[
            pl.BlockSpec((1, 1, _BLK), lambda i: (i, 0, 0)),
            pl.BlockSpec((1, _TPAD), lambda i: (0, 0)),
            pl.BlockSpec((1, _TPAD), lambda i: (0, 0)),
            pl.BlockSpec((_BLK, _COLS), lambda i: (i, 0)),
            pl.BlockSpec((_BLK, _COLS), lambda i: (i, 0)),
        ],
        out_specs=pl.BlockSpec((_BLK, _COLS), lambda i: (i, 0)),
        out_shape=jax.ShapeDtypeStruct((_ROWS, _COLS), jnp.float32),
    )(t32, sa, sb, x0, eps)
